# Initial kernel scaffold; baseline (speedup 1.0000x reference)
import jax
import jax.numpy as jnp
from jax.experimental import pallas as pl


def kernel(t1_embed, str_t2s, t1_pred_context, t2_pred_context, context_features, att_mat, bi_w, bi_b):
    raise NotImplementedError("write your pallas kernel here")


# speedup vs baseline: 2.8623x; 2.8623x over previous
import functools

import jax
import jax.numpy as jnp
from jax import lax
from jax.experimental import pallas as pl
from jax.experimental.pallas import tpu as pltpu
from jax.experimental.pallas import tpu_sc as plsc

VOCAB = 1000000
DIM = 64
CAND = 50
K = 50
STR_DIM = 128
GAMMA = 0.5

LANES = 128
ROWS = 7816
VPAD = ROWS * LANES
NEG = float("-inf")

_NC, _NS = 2, 16
_NW = _NC * _NS
_B_GATHER = 2560
_B_PER_W = _B_GATHER // _NW


def _topk_body(x_ref, out_ref, bufv, bufi, *, max_passes):
    rows_i = lax.broadcasted_iota(jnp.int32, (ROWS, LANES), 0)
    lane_i = lax.broadcasted_iota(jnp.int32, (ROWS, LANES), 1)
    lane_row = lax.broadcasted_iota(jnp.int32, (1, LANES), 1)

    bufv[...] = jnp.full((max_passes, LANES), NEG, jnp.float32)
    bufi[...] = jnp.zeros((max_passes, LANES), jnp.int32)

    def cond(carry):
        p, cnt = carry
        return jnp.logical_and(cnt < K, p < max_passes)

    def body(carry):
        p, _ = carry
        x = x_ref[0]
        m = jnp.max(x, axis=0, keepdims=True)
        eq = x == m
        r = jnp.min(jnp.where(eq, rows_i, jnp.int32(2**30)),
                    axis=0, keepdims=True)
        hit = rows_i == r
        x_new = jnp.where(hit, NEG, x)
        x_ref[0] = x_new
        vidx = r * LANES + lane_row
        bufv[pl.ds(p, 1), :] = m
        bufi[pl.ds(p, 1), :] = vidx
        max_rem = jnp.max(x_new)
        cnt = jnp.sum((bufv[...] > max_rem).astype(jnp.int32))
        return p + 1, cnt

    lax.while_loop(cond, body, (jnp.int32(0), jnp.int32(0)))

    def merge(k, acc):
        bv = bufv[...]
        m = jnp.max(bv)
        eq = bv == m
        sel = jnp.min(jnp.where(eq, bufi[...], jnp.int32(2**30)))
        bufv[...] = jnp.where(jnp.logical_and(eq, bufi[...] == sel), NEG, bv)
        out_lane = lax.broadcasted_iota(jnp.int32, (1, 64), 1)
        return jnp.where(out_lane == k, sel, acc)

    acc = lax.fori_loop(0, K, merge, jnp.zeros((1, 64), jnp.int32))
    out_ref[pl.ds(pl.program_id(0), 1), :] = acc


def _topk_call(x3):
    n = x3.shape[0]
    mp = 56
    return pl.pallas_call(
        functools.partial(_topk_body, max_passes=mp),
        grid=(n,),
        in_specs=[pl.BlockSpec((1, ROWS, LANES), lambda i: (i, 0, 0))],
        out_specs=pl.BlockSpec((n, 64), lambda i: (0, 0)),
        out_shape=jax.ShapeDtypeStruct((n, 64), jnp.int32),
        scratch_shapes=[
            pltpu.VMEM((mp, LANES), jnp.float32),
            pltpu.VMEM((mp, LANES), jnp.int32),
        ],
    )(x3)


def _gather(table, idx):
    mesh = plsc.VectorSubcoreMesh(core_axis_name="c", subcore_axis_name="s")

    @functools.partial(
        pl.kernel,
        mesh=mesh,
        out_type=jax.ShapeDtypeStruct((_B_GATHER, 2 * DIM), jnp.float32),
        scratch_types=[
            pltpu.VMEM((_B_PER_W,), jnp.int32),
            pltpu.VMEM((_B_PER_W, 2 * DIM), jnp.float32),
            pltpu.SemaphoreType.DMA,
        ],
    )
    def k(table_hbm, idx_hbm, out_hbm, idx_v, rows_v, sem):
        wid = lax.axis_index("s") * _NC + lax.axis_index("c")
        base = wid * _B_PER_W
        pltpu.sync_copy(idx_hbm.at[pl.ds(base, _B_PER_W)], idx_v)
        pltpu.async_copy(table_hbm.at[idx_v], rows_v, sem).wait()
        pltpu.sync_copy(rows_v, out_hbm.at[pl.ds(base, _B_PER_W)])

    return k(table, idx)


def _score_body(g_ref, par_ref, att_ref, bw_ref, bb_ref, t1e_ref, st2_ref,
                out_ref):
    c = pl.program_id(0)
    W = 2 * DIM
    ii = lax.broadcasted_iota(jnp.int32, (W, W), 0)
    jj = lax.broadcasted_iota(jnp.int32, (W, W), 1)
    swap = jnp.where(ii < DIM, ii + DIM, ii - DIM)
    P = (jj == swap).astype(jnp.float32)
    lmask = (lax.broadcasted_iota(jnp.int32, (1, W), 1) < DIM
             ).astype(jnp.float32)

    def pick(raw, par):
        sw = jnp.dot(raw, P, preferred_element_type=jnp.float32)
        return jnp.where(par == 1, sw, raw) * lmask

    t1c = pick(g_ref[0:K, :], par_ref[0:K, :])
    t2c = pick(g_ref[pl.ds(K + c * K, K), :],
               par_ref[pl.ds(K + c * K, K), :])
    p1 = jnp.dot(t1c, att_ref[...], preferred_element_type=jnp.float32)
    sim = jnp.tanh(lax.dot_general(
        p1, t2c, (((1,), (1,)), ((), ())),
        preferred_element_type=jnp.float32))

    rm = jnp.mean(sim, axis=1, keepdims=True)
    re = jnp.exp(rm - jnp.max(rm))
    rs = re / jnp.sum(re)
    cm = jnp.mean(sim, axis=0, keepdims=True)
    ce = jnp.exp(cm - jnp.max(cm))
    cs = ce / jnp.sum(ce)

    new_a = jnp.sum(t1c * rs, axis=0, keepdims=True)
    new_b = jnp.dot(cs, t2c, preferred_element_type=jnp.float32)
    con = jnp.sum(jnp.dot(new_a, bw_ref[...],
                          preferred_element_type=jnp.float32) * new_b)
    con = con + bb_ref[0, 0]

    a = t1e_ref[...]
    b = st2_ref[pl.ds(c, 1), :]
    num = jnp.sum(a * b)
    den = jnp.sqrt(jnp.sum(a * a)) * jnp.sqrt(jnp.sum(b * b))
    strs = num / jnp.maximum(den, jnp.float32(1e-8))

    y = (1.0 - GAMMA) * strs + GAMMA * con
    out_ref[pl.ds(c, 1), :] = jnp.full((1, 1), 0.0, jnp.float32) + y


def _score_call(g, par, att, bw, bb2, t1e, st2):
    return pl.pallas_call(
        _score_body,
        grid=(CAND,),
        in_specs=[
            pl.BlockSpec((_B_GATHER, 2 * DIM), lambda c: (0, 0)),
            pl.BlockSpec((_B_GATHER, 1), lambda c: (0, 0)),
            pl.BlockSpec((2 * DIM, 2 * DIM), lambda c: (0, 0)),
            pl.BlockSpec((2 * DIM, 2 * DIM), lambda c: (0, 0)),
            pl.BlockSpec((1, 1), lambda c: (0, 0)),
            pl.BlockSpec((1, STR_DIM), lambda c: (0, 0)),
            pl.BlockSpec((CAND, STR_DIM), lambda c: (0, 0)),
        ],
        out_specs=pl.BlockSpec((CAND, 1), lambda c: (0, 0)),
        out_shape=jax.ShapeDtypeStruct((CAND, 1), jnp.float32),
    )(g, par, att, bw, bb2, t1e, st2)


def kernel(t1_embed, str_t2s, t1_pred_context, t2_pred_context,
           context_features, att_mat, bi_w, bi_b):
    pad = VPAD - VOCAB
    x1 = jnp.pad(t1_pred_context, ((0, 0), (0, pad)),
                 constant_values=-jnp.inf).reshape(1, ROWS, LANES)
    x2 = jnp.pad(t2_pred_context, ((0, 0), (0, pad)),
                 constant_values=-jnp.inf).reshape(CAND, ROWS, LANES)

    idx1 = _topk_call(x1)[:, :K]
    idx2 = _topk_call(x2)[:, :K]

    flat_idx = jnp.concatenate([
        idx1.reshape(-1), idx2.reshape(-1),
        jnp.zeros((_B_GATHER - (CAND + 1) * K,), jnp.int32)])
    table2 = context_features.reshape(VOCAB // 2, 2 * DIM)
    g = _gather(table2, flat_idx // 2)
    par = (flat_idx % 2).astype(jnp.int32).reshape(_B_GATHER, 1)

    zpad = ((0, DIM), (0, DIM))
    attp = jnp.pad(att_mat, zpad)
    bwp = jnp.pad(bi_w, zpad)
    y = _score_call(g, par, attp, bwp, bi_b.reshape(1, 1),
                    t1_embed, str_t2s)
    return y.reshape(1, CAND)


# speedup vs baseline: 2.9648x; 1.0358x over previous
import functools

import jax
import jax.numpy as jnp
from jax import lax
from jax.experimental import pallas as pl
from jax.experimental.pallas import tpu as pltpu
from jax.experimental.pallas import tpu_sc as plsc

VOCAB = 1000000
DIM = 64
CAND = 50
K = 50
STR_DIM = 128
GAMMA = 0.5

LANES = 128
ROWS = 7816
VPAD = ROWS * LANES
NEG = float("-inf")

_NC, _NS = 2, 16
_NW = _NC * _NS
_B_GATHER = 2560
_B_PER_W = _B_GATHER // _NW


def _topk_body(x_ref, out_ref, bufv, bufi, *, max_passes):
    rows_i = lax.broadcasted_iota(jnp.int32, (ROWS, LANES), 0)
    lane_i = lax.broadcasted_iota(jnp.int32, (ROWS, LANES), 1)
    lane_row = lax.broadcasted_iota(jnp.int32, (1, LANES), 1)

    bufv[...] = jnp.full((max_passes, LANES), NEG, jnp.float32)
    bufi[...] = jnp.zeros((max_passes, LANES), jnp.int32)

    def cond(carry):
        p, cnt = carry
        return jnp.logical_and(cnt < K, p < max_passes)

    def body(carry):
        p, _ = carry
        x = x_ref[0]
        m = jnp.max(x, axis=0, keepdims=True)
        gmax = jnp.max(m)
        cnt = jnp.sum((bufv[...] > gmax).astype(jnp.int32))

        @pl.when(cnt < K)
        def _extract():
            eq = x == m
            r = jnp.min(jnp.where(eq, rows_i, jnp.int32(2**30)),
                        axis=0, keepdims=True)
            x_ref[0] = jnp.where(eq, NEG, x)
            bufv[pl.ds(p, 1), :] = m
            bufi[pl.ds(p, 1), :] = r * LANES + lane_row

        return p + 1, cnt

    lax.while_loop(cond, body, (jnp.int32(0), jnp.int32(0)))

    def merge(k, acc):
        bv = bufv[...]
        m = jnp.max(bv)
        eq = bv == m
        sel = jnp.min(jnp.where(eq, bufi[...], jnp.int32(2**30)))
        bufv[...] = jnp.where(jnp.logical_and(eq, bufi[...] == sel), NEG, bv)
        out_lane = lax.broadcasted_iota(jnp.int32, (1, 64), 1)
        return jnp.where(out_lane == k, sel, acc)

    acc = lax.fori_loop(0, K, merge, jnp.zeros((1, 64), jnp.int32))
    out_ref[pl.ds(pl.program_id(0), 1), :] = acc


def _topk_call(x3):
    n = x3.shape[0]
    mp = 56
    return pl.pallas_call(
        functools.partial(_topk_body, max_passes=mp),
        grid=(n,),
        in_specs=[pl.BlockSpec((1, ROWS, LANES), lambda i: (i, 0, 0))],
        out_specs=pl.BlockSpec((n, 64), lambda i: (0, 0)),
        out_shape=jax.ShapeDtypeStruct((n, 64), jnp.int32),
        scratch_shapes=[
            pltpu.VMEM((mp, LANES), jnp.float32),
            pltpu.VMEM((mp, LANES), jnp.int32),
        ],
    )(x3)


def _gather(table, idx):
    mesh = plsc.VectorSubcoreMesh(core_axis_name="c", subcore_axis_name="s")

    @functools.partial(
        pl.kernel,
        mesh=mesh,
        out_type=jax.ShapeDtypeStruct((_B_GATHER, 2 * DIM), jnp.float32),
        scratch_types=[
            pltpu.VMEM((_B_PER_W,), jnp.int32),
            pltpu.VMEM((_B_PER_W, 2 * DIM), jnp.float32),
            pltpu.SemaphoreType.DMA,
        ],
    )
    def k(table_hbm, idx_hbm, out_hbm, idx_v, rows_v, sem):
        wid = lax.axis_index("s") * _NC + lax.axis_index("c")
        base = wid * _B_PER_W
        pltpu.sync_copy(idx_hbm.at[pl.ds(base, _B_PER_W)], idx_v)
        pltpu.async_copy(table_hbm.at[idx_v], rows_v, sem).wait()
        pltpu.sync_copy(rows_v, out_hbm.at[pl.ds(base, _B_PER_W)])

    return k(table, idx)


def _score_body(g_ref, par_ref, att_ref, bw_ref, bb_ref, t1e_ref, st2_ref,
                out_ref):
    c = pl.program_id(0)
    W = 2 * DIM
    ii = lax.broadcasted_iota(jnp.int32, (W, W), 0)
    jj = lax.broadcasted_iota(jnp.int32, (W, W), 1)
    swap = jnp.where(ii < DIM, ii + DIM, ii - DIM)
    P = (jj == swap).astype(jnp.float32)
    lmask = (lax.broadcasted_iota(jnp.int32, (1, W), 1) < DIM
             ).astype(jnp.float32)

    def pick(raw, par):
        sw = jnp.dot(raw, P, preferred_element_type=jnp.float32)
        return jnp.where(par == 1, sw, raw) * lmask

    t1c = pick(g_ref[0:K, :], par_ref[0:K, :])
    t2c = pick(g_ref[pl.ds(K + c * K, K), :],
               par_ref[pl.ds(K + c * K, K), :])
    p1 = jnp.dot(t1c, att_ref[...], preferred_element_type=jnp.float32)
    sim = jnp.tanh(lax.dot_general(
        p1, t2c, (((1,), (1,)), ((), ())),
        preferred_element_type=jnp.float32))

    rm = jnp.mean(sim, axis=1, keepdims=True)
    re = jnp.exp(rm - jnp.max(rm))
    rs = re / jnp.sum(re)
    cm = jnp.mean(sim, axis=0, keepdims=True)
    ce = jnp.exp(cm - jnp.max(cm))
    cs = ce / jnp.sum(ce)

    new_a = jnp.sum(t1c * rs, axis=0, keepdims=True)
    new_b = jnp.dot(cs, t2c, preferred_element_type=jnp.float32)
    con = jnp.sum(jnp.dot(new_a, bw_ref[...],
                          preferred_element_type=jnp.float32) * new_b)
    con = con + bb_ref[0, 0]

    a = t1e_ref[...]
    b = st2_ref[pl.ds(c, 1), :]
    num = jnp.sum(a * b)
    den = jnp.sqrt(jnp.sum(a * a)) * jnp.sqrt(jnp.sum(b * b))
    strs = num / jnp.maximum(den, jnp.float32(1e-8))

    y = (1.0 - GAMMA) * strs + GAMMA * con
    out_ref[pl.ds(c, 1), :] = jnp.full((1, 1), 0.0, jnp.float32) + y


def _score_call(g, par, att, bw, bb2, t1e, st2):
    return pl.pallas_call(
        _score_body,
        grid=(CAND,),
        in_specs=[
            pl.BlockSpec((_B_GATHER, 2 * DIM), lambda c: (0, 0)),
            pl.BlockSpec((_B_GATHER, 1), lambda c: (0, 0)),
            pl.BlockSpec((2 * DIM, 2 * DIM), lambda c: (0, 0)),
            pl.BlockSpec((2 * DIM, 2 * DIM), lambda c: (0, 0)),
            pl.BlockSpec((1, 1), lambda c: (0, 0)),
            pl.BlockSpec((1, STR_DIM), lambda c: (0, 0)),
            pl.BlockSpec((CAND, STR_DIM), lambda c: (0, 0)),
        ],
        out_specs=pl.BlockSpec((CAND, 1), lambda c: (0, 0)),
        out_shape=jax.ShapeDtypeStruct((CAND, 1), jnp.float32),
    )(g, par, att, bw, bb2, t1e, st2)


def kernel(t1_embed, str_t2s, t1_pred_context, t2_pred_context,
           context_features, att_mat, bi_w, bi_b):
    pad = VPAD - VOCAB
    x1 = jnp.pad(t1_pred_context, ((0, 0), (0, pad)),
                 constant_values=-jnp.inf).reshape(1, ROWS, LANES)
    x2 = jnp.pad(t2_pred_context, ((0, 0), (0, pad)),
                 constant_values=-jnp.inf).reshape(CAND, ROWS, LANES)

    idx1 = _topk_call(x1)[:, :K]
    idx2 = _topk_call(x2)[:, :K]

    flat_idx = jnp.concatenate([
        idx1.reshape(-1), idx2.reshape(-1),
        jnp.zeros((_B_GATHER - (CAND + 1) * K,), jnp.int32)])
    table2 = context_features.reshape(VOCAB // 2, 2 * DIM)
    g = _gather(table2, flat_idx // 2)
    par = (flat_idx % 2).astype(jnp.int32).reshape(_B_GATHER, 1)

    zpad = ((0, DIM), (0, DIM))
    attp = jnp.pad(att_mat, zpad)
    bwp = jnp.pad(bi_w, zpad)
    y = _score_call(g, par, attp, bwp, bi_b.reshape(1, 1),
                    t1_embed, str_t2s)
    return y.reshape(1, CAND)
